# Initial kernel scaffold; baseline (speedup 1.0000x reference)
import jax
import jax.numpy as jnp
from jax.experimental import pallas as pl


def kernel(x, edge_index, W1, b1, W2, b2):
    raise NotImplementedError("write your pallas kernel here")


# speedup vs baseline: 8.6582x; 8.6582x over previous
import functools

import jax
import jax.numpy as jnp
from jax import lax
from jax.experimental import pallas as pl
from jax.experimental.pallas import tpu as pltpu
from jax.experimental.pallas import tpu_sc as plsc

N = 10000
E = 320000
D = 128
NC = 2
NS = 16
NW = NC * NS

HR = 640
EPT = E // NW
DEG_G = EPT // 16

CH = 128
K = -(-E // (NW * CH))
PADE = NW * K * CH
R = HR * 16

_MESH = plsc.VectorSubcoreMesh(core_axis_name="c", subcore_axis_name="s")
_SC_PARAMS = pltpu.CompilerParams(needs_layout_passes=False,
                                  use_tc_tiling_on_sc=False)


HF = HR * 16


def _deg_body(src_hbm, dst_hbm, out_hbm, src_v, dst_v, hs_v, hd_v):
    c = lax.axis_index("c")
    s = lax.axis_index("s")
    w = s * NC + c

    z = jnp.zeros((16,), jnp.float32)

    def zero_row(i, carry):
        hs_v[pl.ds(i * 16, 16)] = z
        hd_v[pl.ds(i * 16, 16)] = z
        return carry

    lax.fori_loop(0, HR, zero_row, 0)

    pltpu.sync_copy(src_hbm.at[w], src_v)
    pltpu.sync_copy(dst_hbm.at[w], dst_v)

    ones = jnp.ones((16,), jnp.float32)

    def scat(i, carry):
        plsc.addupdate_scatter(hs_v, [src_v[i, :]], ones)
        plsc.addupdate_scatter(hd_v, [dst_v[i, :]], ones)
        return carry

    lax.fori_loop(0, DEG_G, scat, 0)

    pltpu.sync_copy(hs_v, out_hbm.at[w, 0])
    pltpu.sync_copy(hd_v, out_hbm.at[w, 1])


_deg_call = functools.partial(
    pl.kernel,
    out_type=jax.ShapeDtypeStruct((NW, 2, HF), jnp.float32),
    mesh=_MESH,
    compiler_params=_SC_PARAMS,
    scratch_types=[
        pltpu.VMEM((DEG_G, 16), jnp.int32),
        pltpu.VMEM((DEG_G, 16), jnp.int32),
        pltpu.VMEM((HF,), jnp.float32),
        pltpu.VMEM((HF,), jnp.float32),
    ],
)(_deg_body)


def _agg_body(xs_hbm, srcp_hbm, dstp_hbm, zer_hbm, out_hbm,
              src_v, dst_v, buf, sem, agg_sh):
    c = lax.axis_index("c")
    s = lax.axis_index("s")
    w = s * NC + c
    rows = R // NS

    pltpu.sync_copy(zer_hbm.at[pl.ds(s * rows, rows)], agg_sh.at[pl.ds(s * rows, rows)])
    pltpu.sync_copy(srcp_hbm.at[w], src_v)
    pltpu.sync_copy(dstp_hbm.at[w], dst_v)
    plsc.subcore_barrier()

    def step(j, carry):
        pltpu.async_copy(xs_hbm.at[src_v.at[j]], buf, sem).wait()
        pltpu.sync_copy(buf, agg_sh.at[dst_v.at[j]], add=True)
        return carry

    lax.fori_loop(0, K, step, 0)
    plsc.subcore_barrier()

    pltpu.sync_copy(agg_sh.at[pl.ds(s * rows, rows)], out_hbm.at[c, pl.ds(s * rows, rows)])


_agg_call = functools.partial(
    pl.kernel,
    out_type=jax.ShapeDtypeStruct((NC, R, D), jnp.float32),
    mesh=_MESH,
    compiler_params=_SC_PARAMS,
    scratch_types=[
        pltpu.VMEM((K, CH), jnp.int32),
        pltpu.VMEM((K, CH), jnp.int32),
        pltpu.VMEM((CH, D), jnp.float32),
        pltpu.SemaphoreType.DMA,
        pltpu.VMEM_SHARED((R, D), jnp.float32),
    ],
)(_agg_body)


BR = 1000


def _prep_body(x_ref, dsT, ddT, xs_ref, ns_ref, nd_ref):
    ds = jnp.sum(dsT[...], axis=1, keepdims=True)
    dd = jnp.sum(ddT[...], axis=1, keepdims=True)
    ns = jnp.where(ds > 0, lax.rsqrt(jnp.maximum(ds, 1.0)), 0.0)
    nd = jnp.where(dd > 0, lax.rsqrt(jnp.maximum(dd, 1.0)), 0.0)
    xs_ref[...] = x_ref[...] * ns
    ns_ref[...] = ns
    nd_ref[...] = nd


_col = pl.BlockSpec((BR, 1), lambda i: (i, 0))
_rowblk = pl.BlockSpec((BR, D), lambda i: (i, 0))
_degblk = pl.BlockSpec((BR, NW), lambda i: (i, 0))

_prep_call = pl.pallas_call(
    _prep_body,
    grid=(N // BR,),
    in_specs=[_rowblk, _degblk, _degblk],
    out_specs=[_rowblk, _col, _col],
    out_shape=[
        jax.ShapeDtypeStruct((N, D), jnp.float32),
        jax.ShapeDtypeStruct((N, 1), jnp.float32),
        jax.ShapeDtypeStruct((N, 1), jnp.float32),
    ],
)


def _dense_body(a0, a1, nd, ns, w_ref, b_ref, o_ref, *, final):
    g = (a0[...] + a1[...]) * nd[...]
    h = jnp.dot(g, w_ref[...], preferred_element_type=jnp.float32) + b_ref[...]
    h = jnp.maximum(h, 0.0)
    o_ref[...] = h if final else h * ns[...]


def _make_dense(final):
    return pl.pallas_call(
        functools.partial(_dense_body, final=final),
        grid=(N // BR,),
        in_specs=[
            _rowblk, _rowblk, _col, _col,
            pl.BlockSpec((D, D), lambda i: (0, 0)),
            pl.BlockSpec((1, D), lambda i: (0, 0)),
        ],
        out_specs=_rowblk,
        out_shape=jax.ShapeDtypeStruct((N, D), jnp.float32),
    )


_dense_mid = _make_dense(False)
_dense_fin = _make_dense(True)


def kernel(x, edge_index, W1, b1, W2, b2):
    src = edge_index[0].astype(jnp.int32)
    dst = edge_index[1].astype(jnp.int32)

    deg = _deg_call(src.reshape(NW, DEG_G, 16), dst.reshape(NW, DEG_G, 16))
    dsT = deg[:, 0, :N].T
    ddT = deg[:, 1, :N].T
    xs1, ns, nd = _prep_call(x, dsT, ddT)

    pad = PADE - E
    srcp = jnp.concatenate([src, jnp.zeros((pad,), jnp.int32)]).reshape(NW, K, CH)
    dstp = jnp.concatenate([dst, jnp.full((pad,), R - 1, jnp.int32)]).reshape(NW, K, CH)
    zer = jnp.zeros((R, D), jnp.float32)

    agg1 = _agg_call(xs1, srcp, dstp, zer)
    h1s = _dense_mid(agg1[0, :N], agg1[1, :N], nd, ns, W1, b1.reshape(1, D))
    agg2 = _agg_call(h1s, srcp, dstp, zer)
    return _dense_fin(agg2[0, :N], agg2[1, :N], nd, ns, W2, b2.reshape(1, D))


# speedup vs baseline: 16.5432x; 1.9107x over previous
import functools

import jax
import jax.numpy as jnp
from jax import lax
from jax.experimental import pallas as pl
from jax.experimental.pallas import tpu as pltpu
from jax.experimental.pallas import tpu_sc as plsc

N = 10000
E = 320000
D = 128
NC = 2
NS = 16
NW = NC * NS

HR = 640
EPT = E // NW
DEG_G = EPT // 16

CH = 64
NB = 2
EPW = E // NW
K = 160
PW = K * CH - EPW
R = HR * 16

_MESH = plsc.VectorSubcoreMesh(core_axis_name="c", subcore_axis_name="s")
_SC_PARAMS = pltpu.CompilerParams(needs_layout_passes=False,
                                  use_tc_tiling_on_sc=False)


HF = HR * 16


def _deg_body(src_hbm, dst_hbm, out_hbm, src_v, dst_v, hs_v, hd_v):
    c = lax.axis_index("c")
    s = lax.axis_index("s")
    w = s * NC + c

    z = jnp.zeros((16,), jnp.float32)

    def zero_row(i, carry):
        hs_v[pl.ds(i * 16, 16)] = z
        hd_v[pl.ds(i * 16, 16)] = z
        return carry

    lax.fori_loop(0, HR, zero_row, 0)

    pltpu.sync_copy(src_hbm.at[w], src_v)
    pltpu.sync_copy(dst_hbm.at[w], dst_v)

    ones = jnp.ones((16,), jnp.float32)

    def scat(i, carry):
        plsc.addupdate_scatter(hs_v, [src_v[i, :]], ones)
        plsc.addupdate_scatter(hd_v, [dst_v[i, :]], ones)
        return carry

    lax.fori_loop(0, DEG_G, scat, 0)

    pltpu.sync_copy(hs_v, out_hbm.at[w, 0])
    pltpu.sync_copy(hd_v, out_hbm.at[w, 1])


_deg_call = functools.partial(
    pl.kernel,
    out_type=jax.ShapeDtypeStruct((NW, 2, HF), jnp.float32),
    mesh=_MESH,
    compiler_params=_SC_PARAMS,
    scratch_types=[
        pltpu.VMEM((DEG_G, 16), jnp.int32),
        pltpu.VMEM((DEG_G, 16), jnp.int32),
        pltpu.VMEM((HF,), jnp.float32),
        pltpu.VMEM((HF,), jnp.float32),
    ],
)(_deg_body)


def _agg_body(xs_hbm, srcp_hbm, dstp_hbm, zer_hbm, out_hbm,
              src_v, dst_v, bufs, sem0, sem1, agg_sh):
    c = lax.axis_index("c")
    s = lax.axis_index("s")
    w = s * NC + c
    rows = R // NS
    sems = [sem0, sem1]

    pltpu.sync_copy(zer_hbm.at[pl.ds(s * rows, rows)], agg_sh.at[pl.ds(s * rows, rows)])
    pltpu.sync_copy(srcp_hbm.at[w], src_v)
    pltpu.sync_copy(dstp_hbm.at[w], dst_v)
    plsc.subcore_barrier()

    for b in range(NB):
        pltpu.async_copy(xs_hbm.at[src_v.at[b]], bufs.at[b], sems[b])

    def outer(g, carry):
        for b in range(NB):
            j = g * NB + b
            pltpu.make_async_copy(xs_hbm.at[src_v.at[j]], bufs.at[b], sems[b]).wait()
            pltpu.sync_copy(bufs.at[b], agg_sh.at[dst_v.at[j]], add=True)

            @pl.when(j + NB < K)
            def _():
                pltpu.async_copy(xs_hbm.at[src_v.at[j + NB]], bufs.at[b], sems[b])

        return carry

    lax.fori_loop(0, K // NB, outer, 0)
    plsc.subcore_barrier()

    pltpu.sync_copy(agg_sh.at[pl.ds(s * rows, rows)], out_hbm.at[c, pl.ds(s * rows, rows)])


_agg_call = functools.partial(
    pl.kernel,
    out_type=jax.ShapeDtypeStruct((NC, R, D), jnp.float32),
    mesh=_MESH,
    compiler_params=_SC_PARAMS,
    scratch_types=[
        pltpu.VMEM((K, CH), jnp.int32),
        pltpu.VMEM((K, CH), jnp.int32),
        pltpu.VMEM((NB, CH, D), jnp.float32),
        pltpu.SemaphoreType.DMA,
        pltpu.SemaphoreType.DMA,
        pltpu.VMEM_SHARED((R, D), jnp.float32),
    ],
)(_agg_body)


BR = 1000


def _prep_body(x_ref, dsT, ddT, xs_ref, ns_ref, nd_ref):
    ds = jnp.sum(dsT[...], axis=1, keepdims=True)
    dd = jnp.sum(ddT[...], axis=1, keepdims=True)
    ns = jnp.where(ds > 0, lax.rsqrt(jnp.maximum(ds, 1.0)), 0.0)
    nd = jnp.where(dd > 0, lax.rsqrt(jnp.maximum(dd, 1.0)), 0.0)
    xs_ref[...] = x_ref[...] * ns
    ns_ref[...] = ns
    nd_ref[...] = nd


_col = pl.BlockSpec((BR, 1), lambda i: (i, 0))
_rowblk = pl.BlockSpec((BR, D), lambda i: (i, 0))
_degblk = pl.BlockSpec((BR, NW), lambda i: (i, 0))

_prep_call = pl.pallas_call(
    _prep_body,
    grid=(N // BR,),
    in_specs=[_rowblk, _degblk, _degblk],
    out_specs=[_rowblk, _col, _col],
    out_shape=[
        jax.ShapeDtypeStruct((N, D), jnp.float32),
        jax.ShapeDtypeStruct((N, 1), jnp.float32),
        jax.ShapeDtypeStruct((N, 1), jnp.float32),
    ],
)


def _dense_body(a0, a1, nd, ns, w_ref, b_ref, o_ref, *, final):
    g = (a0[...] + a1[...]) * nd[...]
    h = jnp.dot(g, w_ref[...], preferred_element_type=jnp.float32) + b_ref[...]
    h = jnp.maximum(h, 0.0)
    o_ref[...] = h if final else h * ns[...]


def _make_dense(final):
    return pl.pallas_call(
        functools.partial(_dense_body, final=final),
        grid=(N // BR,),
        in_specs=[
            _rowblk, _rowblk, _col, _col,
            pl.BlockSpec((D, D), lambda i: (0, 0)),
            pl.BlockSpec((1, D), lambda i: (0, 0)),
        ],
        out_specs=_rowblk,
        out_shape=jax.ShapeDtypeStruct((N, D), jnp.float32),
    )


_dense_mid = _make_dense(False)
_dense_fin = _make_dense(True)


def kernel(x, edge_index, W1, b1, W2, b2):
    src = edge_index[0].astype(jnp.int32)
    dst = edge_index[1].astype(jnp.int32)

    deg = _deg_call(src.reshape(NW, DEG_G, 16), dst.reshape(NW, DEG_G, 16))
    dsT = deg[:, 0, :N].T
    ddT = deg[:, 1, :N].T
    xs1, ns, nd = _prep_call(x, dsT, ddT)

    dump = jnp.broadcast_to(jnp.arange(PW, dtype=jnp.int32)[None], (NW, PW))
    srcp = jnp.concatenate([src.reshape(NW, EPW), dump], axis=1).reshape(NW, K, CH)
    dstp = jnp.concatenate([dst.reshape(NW, EPW), dump + N], axis=1).reshape(NW, K, CH)
    zer = jnp.zeros((R, D), jnp.float32)

    agg1 = _agg_call(xs1, srcp, dstp, zer)
    h1s = _dense_mid(agg1[0, :N], agg1[1, :N], nd, ns, W1, b1.reshape(1, D))
    agg2 = _agg_call(h1s, srcp, dstp, zer)
    return _dense_fin(agg2[0, :N], agg2[1, :N], nd, ns, W2, b2.reshape(1, D))


# speedup vs baseline: 18.8154x; 1.1374x over previous
import functools

import jax
import jax.numpy as jnp
from jax import lax
from jax.experimental import pallas as pl
from jax.experimental.pallas import tpu as pltpu
from jax.experimental.pallas import tpu_sc as plsc

N = 10000
E = 320000
D = 128
NC = 2
NS = 16
NW = NC * NS

HR = 640
EPT = E // NW
DEG_G = EPT // 16

CH = 64
NB = 3
EPW = E // NW
K = 159
PW = K * CH - EPW
R = HR * 16

_MESH = plsc.VectorSubcoreMesh(core_axis_name="c", subcore_axis_name="s")
_SC_PARAMS = pltpu.CompilerParams(needs_layout_passes=False,
                                  use_tc_tiling_on_sc=False)


HF = HR * 16


def _deg_body(src_hbm, dst_hbm, out_hbm, src_v, dst_v, hs_v, hd_v):
    c = lax.axis_index("c")
    s = lax.axis_index("s")
    w = s * NC + c

    z = jnp.zeros((16,), jnp.float32)

    def zero_row(i, carry):
        hs_v[pl.ds(i * 16, 16)] = z
        hd_v[pl.ds(i * 16, 16)] = z
        return carry

    lax.fori_loop(0, HR, zero_row, 0)

    pltpu.sync_copy(src_hbm.at[w], src_v)
    pltpu.sync_copy(dst_hbm.at[w], dst_v)

    ones = jnp.ones((16,), jnp.float32)

    def scat(i, carry):
        plsc.addupdate_scatter(hs_v, [src_v[i, :]], ones)
        plsc.addupdate_scatter(hd_v, [dst_v[i, :]], ones)
        return carry

    lax.fori_loop(0, DEG_G, scat, 0)

    pltpu.sync_copy(hs_v, out_hbm.at[w, 0])
    pltpu.sync_copy(hd_v, out_hbm.at[w, 1])


_deg_call = functools.partial(
    pl.kernel,
    out_type=jax.ShapeDtypeStruct((NW, 2, HF), jnp.float32),
    mesh=_MESH,
    compiler_params=_SC_PARAMS,
    scratch_types=[
        pltpu.VMEM((DEG_G, 16), jnp.int32),
        pltpu.VMEM((DEG_G, 16), jnp.int32),
        pltpu.VMEM((HF,), jnp.float32),
        pltpu.VMEM((HF,), jnp.float32),
    ],
)(_deg_body)


def _agg_body(xs_hbm, srcp_hbm, dstp_hbm, zer_hbm, out_hbm,
              src_v, dst_v, bufs, g0, g1, g2, s0, s1, s2, agg_sh):
    c = lax.axis_index("c")
    s = lax.axis_index("s")
    w = s * NC + c
    rows = R // NS
    sems_g = [g0, g1, g2]
    sems_s = [s0, s1, s2]

    pltpu.sync_copy(zer_hbm.at[pl.ds(s * rows, rows)], agg_sh.at[pl.ds(s * rows, rows)])
    pltpu.sync_copy(srcp_hbm.at[w], src_v)
    pltpu.sync_copy(dstp_hbm.at[w], dst_v)
    plsc.subcore_barrier()

    def gather(t, b):
        pltpu.async_copy(xs_hbm.at[src_v.at[t]], bufs.at[b], sems_g[b])

    def scat_wait(t, b):
        pltpu.make_async_copy(bufs.at[b], agg_sh.at[dst_v.at[t]], sems_s[b]).wait()

    gather(0, 0)
    gather(1, 1)

    def outer(g, carry):
        for b in range(NB):
            t = g * NB + b
            pltpu.make_async_copy(xs_hbm.at[src_v.at[t]], bufs.at[b], sems_g[b]).wait()
            pltpu.async_copy(bufs.at[b], agg_sh.at[dst_v.at[t]], sems_s[b], add=True)
            b2 = (b + 2) % NB

            @pl.when(t >= 1)
            def _():
                scat_wait(t - 1, b2)

            @pl.when(t + 2 < K)
            def _():
                gather(t + 2, b2)

        return carry

    lax.fori_loop(0, K // NB, outer, 0)
    scat_wait(K - 1, (K - 1) % NB)
    plsc.subcore_barrier()

    orows = N // NS
    pltpu.sync_copy(agg_sh.at[pl.ds(s * orows, orows)], out_hbm.at[c, pl.ds(s * orows, orows)])


_agg_call = functools.partial(
    pl.kernel,
    out_type=jax.ShapeDtypeStruct((NC, N, D), jnp.float32),
    mesh=_MESH,
    compiler_params=_SC_PARAMS,
    scratch_types=[
        pltpu.VMEM((K, CH), jnp.int32),
        pltpu.VMEM((K, CH), jnp.int32),
        pltpu.VMEM((NB, CH, D), jnp.float32),
        pltpu.SemaphoreType.DMA,
        pltpu.SemaphoreType.DMA,
        pltpu.SemaphoreType.DMA,
        pltpu.SemaphoreType.DMA,
        pltpu.SemaphoreType.DMA,
        pltpu.SemaphoreType.DMA,
        pltpu.VMEM_SHARED((R, D), jnp.float32),
    ],
)(_agg_body)


BR = 1000


def _prep_body(x_ref, dsT, ddT, xs_ref, ns_ref, nd_ref):
    ds = jnp.sum(dsT[...], axis=1, keepdims=True)
    dd = jnp.sum(ddT[...], axis=1, keepdims=True)
    ns = jnp.where(ds > 0, lax.rsqrt(jnp.maximum(ds, 1.0)), 0.0)
    nd = jnp.where(dd > 0, lax.rsqrt(jnp.maximum(dd, 1.0)), 0.0)
    xs_ref[...] = x_ref[...] * ns
    ns_ref[...] = ns
    nd_ref[...] = nd


_col = pl.BlockSpec((BR, 1), lambda i: (i, 0))
_rowblk = pl.BlockSpec((BR, D), lambda i: (i, 0))
_degblk = pl.BlockSpec((BR, NW), lambda i: (i, 0))

_prep_call = pl.pallas_call(
    _prep_body,
    grid=(N // BR,),
    in_specs=[_rowblk, _degblk, _degblk],
    out_specs=[_rowblk, _col, _col],
    out_shape=[
        jax.ShapeDtypeStruct((N, D), jnp.float32),
        jax.ShapeDtypeStruct((N, 1), jnp.float32),
        jax.ShapeDtypeStruct((N, 1), jnp.float32),
    ],
)


def _dense_body(a0, a1, nd, ns, w_ref, b_ref, o_ref, *, final):
    g = (a0[...] + a1[...]) * nd[...]
    h = jnp.dot(g, w_ref[...], preferred_element_type=jnp.float32) + b_ref[...]
    h = jnp.maximum(h, 0.0)
    o_ref[...] = h if final else h * ns[...]


def _make_dense(final):
    return pl.pallas_call(
        functools.partial(_dense_body, final=final),
        grid=(N // BR,),
        in_specs=[
            _rowblk, _rowblk, _col, _col,
            pl.BlockSpec((D, D), lambda i: (0, 0)),
            pl.BlockSpec((1, D), lambda i: (0, 0)),
        ],
        out_specs=_rowblk,
        out_shape=jax.ShapeDtypeStruct((N, D), jnp.float32),
    )


_dense_mid = _make_dense(False)
_dense_fin = _make_dense(True)


def kernel(x, edge_index, W1, b1, W2, b2):
    src = edge_index[0].astype(jnp.int32)
    dst = edge_index[1].astype(jnp.int32)

    deg = _deg_call(src.reshape(NW, DEG_G, 16), dst.reshape(NW, DEG_G, 16))
    dsT = deg[:, 0, :N].T
    ddT = deg[:, 1, :N].T
    xs1, ns, nd = _prep_call(x, dsT, ddT)

    dump = jnp.broadcast_to(jnp.arange(PW, dtype=jnp.int32)[None], (NW, PW))
    srcp = jnp.concatenate([src.reshape(NW, EPW), dump], axis=1).reshape(NW, K, CH)
    dstp = jnp.concatenate([dst.reshape(NW, EPW), dump + N], axis=1).reshape(NW, K, CH)
    zer = jnp.zeros((R, D), jnp.float32)

    agg1 = _agg_call(xs1, srcp, dstp, zer)
    h1s = _dense_mid(agg1[0], agg1[1], nd, ns, W1, b1.reshape(1, D))
    agg2 = _agg_call(h1s, srcp, dstp, zer)
    return _dense_fin(agg2[0], agg2[1], nd, ns, W2, b2.reshape(1, D))


# speedup vs baseline: 19.9443x; 1.0600x over previous
import functools

import jax
import jax.numpy as jnp
from jax import lax
from jax.experimental import pallas as pl
from jax.experimental.pallas import tpu as pltpu
from jax.experimental.pallas import tpu_sc as plsc

N = 10000
E = 320000
D = 128
NC = 2
NS = 16
NW = NC * NS

HR = 640
EPT = E // NW
DEG_G = EPT // 16

CH = 64
NB = 3
EPW = E // NW
K = 159
PW = K * CH - EPW
R = HR * 16

_MESH = plsc.VectorSubcoreMesh(core_axis_name="c", subcore_axis_name="s")
_SC_PARAMS = pltpu.CompilerParams(needs_layout_passes=False,
                                  use_tc_tiling_on_sc=False)


HF = HR * 16
KCH = K * CH


def _deg_body(ei_hbm, hist_out, srcp_out, dstp_out,
              src_v, dst_v, hs_v, hd_v, comb_v, sh_s, sh_d):
    c = lax.axis_index("c")
    s = lax.axis_index("s")
    w = s * NC + c
    rows = HR // NS

    z = jnp.zeros((16,), jnp.float32)

    def zero_row(i, carry):
        hs_v[i, :] = z
        hd_v[i, :] = z
        return carry

    lax.fori_loop(0, HR, zero_row, 0)
    pltpu.sync_copy(hs_v.at[pl.ds(s * rows, rows)], sh_s.at[pl.ds(s * rows, rows)])
    pltpu.sync_copy(hs_v.at[pl.ds(s * rows, rows)], sh_d.at[pl.ds(s * rows, rows)])

    pltpu.sync_copy(ei_hbm.at[0, pl.ds(w * EPW, EPW)], src_v.at[pl.ds(0, EPW)])
    pltpu.sync_copy(ei_hbm.at[1, pl.ds(w * EPW, EPW)], dst_v.at[pl.ds(0, EPW)])
    iota = lax.iota(jnp.int32, 16)
    for g in range(PW // 16):
        src_v[pl.ds(EPW + g * 16, 16)] = iota + (g * 16)
        dst_v[pl.ds(EPW + g * 16, 16)] = iota + (N + g * 16)
    for cc in range(HR // CH):
        for g in range(CH // 16):
            comb_v[cc, pl.ds(g * 16, 16)] = iota + (cc * CH + g * 16)

    ones = jnp.ones((16,), jnp.float32)

    def scat(i, carry):
        i_s = src_v[pl.ds(i * 16, 16)]
        plsc.addupdate_scatter(hs_v, [i_s >> 4, i_s & 15], ones)
        i_d = dst_v[pl.ds(i * 16, 16)]
        plsc.addupdate_scatter(hd_v, [i_d >> 4, i_d & 15], ones)
        return carry

    lax.fori_loop(0, DEG_G, scat, 0)

    pltpu.sync_copy(src_v, srcp_out.at[w])
    pltpu.sync_copy(dst_v, dstp_out.at[w])

    plsc.subcore_barrier()
    for cc in range(HR // CH):
        pltpu.sync_copy(hs_v.at[pl.ds(cc * CH, CH)], sh_s.at[comb_v.at[cc]], add=True)
        pltpu.sync_copy(hd_v.at[pl.ds(cc * CH, CH)], sh_d.at[comb_v.at[cc]], add=True)
    plsc.subcore_barrier()

    pltpu.sync_copy(sh_s.at[pl.ds(s * rows, rows)], hist_out.at[c, 0, pl.ds(s * rows, rows)])
    pltpu.sync_copy(sh_d.at[pl.ds(s * rows, rows)], hist_out.at[c, 1, pl.ds(s * rows, rows)])


_deg_call = functools.partial(
    pl.kernel,
    out_type=[
        jax.ShapeDtypeStruct((NC, 2, HR, 16), jnp.float32),
        jax.ShapeDtypeStruct((NW, KCH), jnp.int32),
        jax.ShapeDtypeStruct((NW, KCH), jnp.int32),
    ],
    mesh=_MESH,
    compiler_params=_SC_PARAMS,
    scratch_types=[
        pltpu.VMEM((KCH,), jnp.int32),
        pltpu.VMEM((KCH,), jnp.int32),
        pltpu.VMEM((HR, 16), jnp.float32),
        pltpu.VMEM((HR, 16), jnp.float32),
        pltpu.VMEM((HR // CH, CH), jnp.int32),
        pltpu.VMEM_SHARED((HR, 16), jnp.float32),
        pltpu.VMEM_SHARED((HR, 16), jnp.float32),
    ],
)(_deg_body)


def _agg_body(xs_hbm, srcp_hbm, dstp_hbm, zer_hbm, out_hbm,
              src_v, dst_v, bufs, g0, g1, g2, s0, s1, s2, agg_sh):
    c = lax.axis_index("c")
    s = lax.axis_index("s")
    w = s * NC + c
    rows = R // NS
    sems_g = [g0, g1, g2]
    sems_s = [s0, s1, s2]

    pltpu.sync_copy(zer_hbm.at[pl.ds(s * rows, rows)], agg_sh.at[pl.ds(s * rows, rows)])
    pltpu.sync_copy(srcp_hbm.at[w], src_v)
    pltpu.sync_copy(dstp_hbm.at[w], dst_v)
    plsc.subcore_barrier()

    def gather(t, b):
        pltpu.async_copy(xs_hbm.at[src_v.at[t]], bufs.at[b], sems_g[b])

    def scat_wait(t, b):
        pltpu.make_async_copy(bufs.at[b], agg_sh.at[dst_v.at[t]], sems_s[b]).wait()

    gather(0, 0)
    gather(1, 1)

    def outer(g, carry):
        for b in range(NB):
            t = g * NB + b
            pltpu.make_async_copy(xs_hbm.at[src_v.at[t]], bufs.at[b], sems_g[b]).wait()
            pltpu.async_copy(bufs.at[b], agg_sh.at[dst_v.at[t]], sems_s[b], add=True)
            b2 = (b + 2) % NB

            @pl.when(t >= 1)
            def _():
                scat_wait(t - 1, b2)

            @pl.when(t + 2 < K)
            def _():
                gather(t + 2, b2)

        return carry

    lax.fori_loop(0, K // NB, outer, 0)
    scat_wait(K - 1, (K - 1) % NB)
    plsc.subcore_barrier()

    orows = N // NS
    pltpu.sync_copy(agg_sh.at[pl.ds(s * orows, orows)], out_hbm.at[c, pl.ds(s * orows, orows)])


_agg_call = functools.partial(
    pl.kernel,
    out_type=jax.ShapeDtypeStruct((NC, N, D), jnp.float32),
    mesh=_MESH,
    compiler_params=_SC_PARAMS,
    scratch_types=[
        pltpu.VMEM((K, CH), jnp.int32),
        pltpu.VMEM((K, CH), jnp.int32),
        pltpu.VMEM((NB, CH, D), jnp.float32),
        pltpu.SemaphoreType.DMA,
        pltpu.SemaphoreType.DMA,
        pltpu.SemaphoreType.DMA,
        pltpu.SemaphoreType.DMA,
        pltpu.SemaphoreType.DMA,
        pltpu.SemaphoreType.DMA,
        pltpu.VMEM_SHARED((R, D), jnp.float32),
    ],
)(_agg_body)


BR = 1000


def _prep_body(x_ref, dsT, ddT, xs_ref, ns_ref, nd_ref):
    ds = jnp.sum(dsT[...], axis=1, keepdims=True)
    dd = jnp.sum(ddT[...], axis=1, keepdims=True)
    ns = jnp.where(ds > 0, lax.rsqrt(jnp.maximum(ds, 1.0)), 0.0)
    nd = jnp.where(dd > 0, lax.rsqrt(jnp.maximum(dd, 1.0)), 0.0)
    xs_ref[...] = x_ref[...] * ns
    ns_ref[...] = ns
    nd_ref[...] = nd


_col = pl.BlockSpec((BR, 1), lambda i: (i, 0))
_rowblk = pl.BlockSpec((BR, D), lambda i: (i, 0))
_degblk = pl.BlockSpec((BR, NC), lambda i: (i, 0))

_prep_call = pl.pallas_call(
    _prep_body,
    grid=(N // BR,),
    in_specs=[_rowblk, _degblk, _degblk],
    out_specs=[_rowblk, _col, _col],
    out_shape=[
        jax.ShapeDtypeStruct((N, D), jnp.float32),
        jax.ShapeDtypeStruct((N, 1), jnp.float32),
        jax.ShapeDtypeStruct((N, 1), jnp.float32),
    ],
)


def _dense_body(a_ref, nd, ns, w_ref, b_ref, o_ref, *, final):
    g = (a_ref[0] + a_ref[1]) * nd[...]
    h = jnp.dot(g, w_ref[...], preferred_element_type=jnp.float32) + b_ref[...]
    h = jnp.maximum(h, 0.0)
    o_ref[...] = h if final else h * ns[...]


def _make_dense(final):
    return pl.pallas_call(
        functools.partial(_dense_body, final=final),
        grid=(N // BR,),
        in_specs=[
            pl.BlockSpec((NC, BR, D), lambda i: (0, i, 0)),
            _col, _col,
            pl.BlockSpec((D, D), lambda i: (0, 0)),
            pl.BlockSpec((1, D), lambda i: (0, 0)),
        ],
        out_specs=_rowblk,
        out_shape=jax.ShapeDtypeStruct((N, D), jnp.float32),
    )


_dense_mid = _make_dense(False)
_dense_fin = _make_dense(True)


def kernel(x, edge_index, W1, b1, W2, b2):
    hist, srcp_f, dstp_f = _deg_call(edge_index.astype(jnp.int32))
    degf = hist.reshape(NC, 2, HF)
    dsT = degf[:, 0, :N].T
    ddT = degf[:, 1, :N].T
    xs1, ns, nd = _prep_call(x, dsT, ddT)

    srcp = srcp_f.reshape(NW, K, CH)
    dstp = dstp_f.reshape(NW, K, CH)
    zer = jnp.zeros((R, D), jnp.float32)

    agg1 = _agg_call(xs1, srcp, dstp, zer)
    h1s = _dense_mid(agg1, nd, ns, W1, b1.reshape(1, D))
    agg2 = _agg_call(h1s, srcp, dstp, zer)
    return _dense_fin(agg2, nd, ns, W2, b2.reshape(1, D))


# speedup vs baseline: 20.9055x; 1.0482x over previous
import functools

import jax
import jax.numpy as jnp
from jax import lax
from jax.experimental import pallas as pl
from jax.experimental.pallas import tpu as pltpu
from jax.experimental.pallas import tpu_sc as plsc

N = 10000
E = 320000
D = 128
NC = 2
NS = 16
NW = NC * NS
NP = 10240

HR = NP // 16
EPW = E // NW
EPT = E // NS
DEG_G = EPT // 16

CH = 64
NB = 3
K = 159
KCH = K * CH
PW = KCH - EPW

_MESH = plsc.VectorSubcoreMesh(core_axis_name="c", subcore_axis_name="s")
_SC_PARAMS = pltpu.CompilerParams(needs_layout_passes=False,
                                  use_tc_tiling_on_sc=False)


def _deg_body(ei_hbm, norm_s_out, norm_d_out, srcp_out, dstp_out,
              id_v, h_v, comb_v, deg_v, nrm_v, dump_v, sh):
    c = lax.axis_index("c")
    s = lax.axis_index("s")
    rows = HR // NS
    iota = lax.iota(jnp.int32, 16)
    z = jnp.zeros((16,), jnp.float32)

    def zero_row(i, carry):
        h_v[i, :] = z
        return carry

    lax.fori_loop(0, HR, zero_row, 0)
    pltpu.sync_copy(h_v.at[pl.ds(s * rows, rows)], sh.at[pl.ds(s * rows, rows)])

    @pl.when(c == 0)
    def _():
        pltpu.sync_copy(ei_hbm.at[0, pl.ds(s * EPT, EPT)], id_v)

    @pl.when(c == 1)
    def _():
        pltpu.sync_copy(ei_hbm.at[1, pl.ds(s * EPT, EPT)], id_v)
    for cc in range(HR // 128):
        for g in range(128 // 16):
            comb_v[cc, pl.ds(g * 16, 16)] = iota + (cc * 128 + g * 16)

    ones = jnp.ones((16,), jnp.float32)

    def scat(i, carry):
        ids = id_v[pl.ds(i * 16, 16)]
        plsc.addupdate_scatter(h_v, [ids >> 4, ids & 15], ones)
        return carry

    lax.fori_loop(0, DEG_G, scat, 0)

    @pl.when(c == 0)
    def _():
        for g in range(PW // 16):
            dump_v[pl.ds(g * 16, 16)] = iota + g * 16
        for half in range(2):
            wv = 2 * s + half
            pltpu.sync_copy(id_v.at[pl.ds(half * EPW, EPW)],
                            srcp_out.at[wv, pl.ds(0, EPW)])
            pltpu.sync_copy(dump_v, srcp_out.at[wv, pl.ds(EPW, PW)])

    @pl.when(c == 1)
    def _():
        for g in range(PW // 16):
            dump_v[pl.ds(g * 16, 16)] = iota + (N + g * 16)
        for half in range(2):
            wv = 2 * s + half
            pltpu.sync_copy(id_v.at[pl.ds(half * EPW, EPW)],
                            dstp_out.at[wv, pl.ds(0, EPW)])
            pltpu.sync_copy(dump_v, dstp_out.at[wv, pl.ds(EPW, PW)])

    plsc.subcore_barrier()
    for cc in range(HR // 128):
        pltpu.sync_copy(h_v.at[pl.ds(cc * 128, 128)], sh.at[comb_v.at[cc]], add=True)
    plsc.subcore_barrier()

    pltpu.sync_copy(sh.at[pl.ds(s * rows, rows)], deg_v)
    half3 = jnp.full((16,), 1.5, jnp.float32)

    def nrm(i, carry):
        dg = deg_v[i, :]
        d = jnp.maximum(dg, 1.0)
        y = plsc.bitcast(jnp.full((16,), 0x5F3759DF, jnp.int32) -
                         (plsc.bitcast(d, jnp.int32) >> 1), jnp.float32)
        hd = 0.5 * d
        y = y * (half3 - hd * y * y)
        y = y * (half3 - hd * y * y)
        y = y * (half3 - hd * y * y)
        nrm_v[pl.ds(i * 16, 16)] = jnp.where(dg > 0, y, 0.0)
        return carry

    lax.fori_loop(0, rows, nrm, 0)

    @pl.when(c == 0)
    def _():
        pltpu.sync_copy(nrm_v, norm_s_out.at[pl.ds(s * (16 * rows), 16 * rows)])

    @pl.when(c == 1)
    def _():
        pltpu.sync_copy(nrm_v, norm_d_out.at[pl.ds(s * (16 * rows), 16 * rows)])


_deg_call = functools.partial(
    pl.kernel,
    out_type=[
        jax.ShapeDtypeStruct((NP,), jnp.float32),
        jax.ShapeDtypeStruct((NP,), jnp.float32),
        jax.ShapeDtypeStruct((NW, KCH), jnp.int32),
        jax.ShapeDtypeStruct((NW, KCH), jnp.int32),
    ],
    mesh=_MESH,
    compiler_params=_SC_PARAMS,
    scratch_types=[
        pltpu.VMEM((EPT,), jnp.int32),
        pltpu.VMEM((HR, 16), jnp.float32),
        pltpu.VMEM((HR // 128, 128), jnp.int32),
        pltpu.VMEM((HR // NS, 16), jnp.float32),
        pltpu.VMEM((16 * (HR // NS),), jnp.float32),
        pltpu.VMEM((PW,), jnp.int32),
        pltpu.VMEM_SHARED((HR, 16), jnp.float32),
    ],
)(_deg_body)


def _agg_body(xs_hbm, srcp_hbm, dstp_hbm, zer_hbm, out_hbm,
              src_v, dst_v, bufs, g0, g1, g2, s0, s1, s2, agg_sh):
    c = lax.axis_index("c")
    s = lax.axis_index("s")
    w = s * NC + c
    rows = NP // NS
    sems_g = [g0, g1, g2]
    sems_s = [s0, s1, s2]

    pltpu.sync_copy(zer_hbm.at[pl.ds(s * rows, rows)], agg_sh.at[pl.ds(s * rows, rows)])
    pltpu.sync_copy(srcp_hbm.at[w], src_v)
    pltpu.sync_copy(dstp_hbm.at[w], dst_v)
    plsc.subcore_barrier()

    def gather(t, b):
        pltpu.async_copy(xs_hbm.at[src_v.at[t]], bufs.at[b], sems_g[b])

    def scat_wait(t, b):
        pltpu.make_async_copy(bufs.at[b], agg_sh.at[dst_v.at[t]], sems_s[b]).wait()

    gather(0, 0)
    gather(1, 1)

    def outer(g, carry):
        for b in range(NB):
            t = g * NB + b
            pltpu.make_async_copy(xs_hbm.at[src_v.at[t]], bufs.at[b], sems_g[b]).wait()
            pltpu.async_copy(bufs.at[b], agg_sh.at[dst_v.at[t]], sems_s[b], add=True)
            b2 = (b + 2) % NB

            @pl.when(t >= 1)
            def _():
                scat_wait(t - 1, b2)

            @pl.when(t + 2 < K)
            def _():
                gather(t + 2, b2)

        return carry

    lax.fori_loop(0, K // NB, outer, 0)
    scat_wait(K - 1, (K - 1) % NB)
    plsc.subcore_barrier()

    pltpu.sync_copy(agg_sh.at[pl.ds(s * rows, rows)], out_hbm.at[c, pl.ds(s * rows, rows)])


_agg_call = functools.partial(
    pl.kernel,
    out_type=jax.ShapeDtypeStruct((NC, NP, D), jnp.float32),
    mesh=_MESH,
    compiler_params=_SC_PARAMS,
    scratch_types=[
        pltpu.VMEM((K, CH), jnp.int32),
        pltpu.VMEM((K, CH), jnp.int32),
        pltpu.VMEM((NB, CH, D), jnp.float32),
        pltpu.SemaphoreType.DMA,
        pltpu.SemaphoreType.DMA,
        pltpu.SemaphoreType.DMA,
        pltpu.SemaphoreType.DMA,
        pltpu.SemaphoreType.DMA,
        pltpu.SemaphoreType.DMA,
        pltpu.VMEM_SHARED((NP, D), jnp.float32),
    ],
)(_agg_body)


BR = 1024


def _prep_body(x_ref, ns_ref, xs_ref):
    xs_ref[...] = x_ref[...] * ns_ref[...]


_col = pl.BlockSpec((BR, 1), lambda i: (i, 0))
_rowblk = pl.BlockSpec((BR, D), lambda i: (i, 0))

_prep_call = pl.pallas_call(
    _prep_body,
    grid=(NP // BR,),
    in_specs=[_rowblk, _col],
    out_specs=_rowblk,
    out_shape=jax.ShapeDtypeStruct((NP, D), jnp.float32),
)


def _dense_body(a_ref, nd, ns, w_ref, b_ref, o_ref, *, final):
    g = (a_ref[0] + a_ref[1]) * nd[...]
    h = jnp.dot(g, w_ref[...], preferred_element_type=jnp.float32) + b_ref[...]
    h = jnp.maximum(h, 0.0)
    o_ref[...] = h if final else h * ns[...]


def _make_dense(final):
    return pl.pallas_call(
        functools.partial(_dense_body, final=final),
        grid=(NP // BR,),
        in_specs=[
            pl.BlockSpec((NC, BR, D), lambda i: (0, i, 0)),
            _col, _col,
            pl.BlockSpec((D, D), lambda i: (0, 0)),
            pl.BlockSpec((1, D), lambda i: (0, 0)),
        ],
        out_specs=_rowblk,
        out_shape=jax.ShapeDtypeStruct((NP, D), jnp.float32),
    )


_dense_mid = _make_dense(False)
_dense_fin = _make_dense(True)


def kernel(x, edge_index, W1, b1, W2, b2):
    ns_f, nd_f, srcp_f, dstp_f = _deg_call(edge_index.astype(jnp.int32))
    ns = ns_f.reshape(NP, 1)
    nd = nd_f.reshape(NP, 1)
    xs1 = _prep_call(x, ns)

    srcp = srcp_f.reshape(NW, K, CH)
    dstp = dstp_f.reshape(NW, K, CH)
    zer = jnp.zeros((NP, D), jnp.float32)

    agg1 = _agg_call(xs1, srcp, dstp, zer)
    h1s = _dense_mid(agg1, nd, ns, W1, b1.reshape(1, D))
    agg2 = _agg_call(h1s, srcp, dstp, zer)
    return _dense_fin(agg2, nd, ns, W2, b2.reshape(1, D))[:N]


# speedup vs baseline: 21.2282x; 1.0154x over previous
import functools

import jax
import jax.numpy as jnp
from jax import lax
from jax.experimental import pallas as pl
from jax.experimental.pallas import tpu as pltpu
from jax.experimental.pallas import tpu_sc as plsc

N = 10000
E = 320000
D = 128
NC = 2
NS = 16
NW = NC * NS
NP = 10240

HR = NP // 16
EPW = E // NW
EPT = E // NS
DEG_G = EPT // 16

CH = 64
NB = 3
K = 159
KCH = K * CH
PW = KCH - EPW

_MESH = plsc.VectorSubcoreMesh(core_axis_name="c", subcore_axis_name="s")
_SC_PARAMS = pltpu.CompilerParams(needs_layout_passes=False,
                                  use_tc_tiling_on_sc=False)


def _deg_body(ei_hbm, norm_s_out, norm_d_out, srcp_out, dstp_out,
              id_v, h_v, comb_v, deg_v, nrm_v, dump_v, sh):
    c = lax.axis_index("c")
    s = lax.axis_index("s")
    rows = HR // NS
    iota = lax.iota(jnp.int32, 16)
    z = jnp.zeros((16,), jnp.float32)

    def zero_row(i, carry):
        h_v[i, :] = z
        return carry

    lax.fori_loop(0, HR, zero_row, 0)
    pltpu.sync_copy(h_v.at[pl.ds(s * rows, rows)], sh.at[pl.ds(s * rows, rows)])

    @pl.when(c == 0)
    def _():
        pltpu.sync_copy(ei_hbm.at[0, pl.ds(s * EPT, EPT)], id_v)

    @pl.when(c == 1)
    def _():
        pltpu.sync_copy(ei_hbm.at[1, pl.ds(s * EPT, EPT)], id_v)
    for cc in range(HR // 128):
        for g in range(128 // 16):
            comb_v[cc, pl.ds(g * 16, 16)] = iota + (cc * 128 + g * 16)

    ones = jnp.ones((16,), jnp.float32)

    def scat(i, carry):
        ids = id_v[pl.ds(i * 16, 16)]
        plsc.addupdate_scatter(h_v, [ids >> 4, ids & 15], ones)
        return carry

    lax.fori_loop(0, DEG_G, scat, 0)

    @pl.when(c == 0)
    def _():
        for g in range(PW // 16):
            dump_v[pl.ds(g * 16, 16)] = iota + g * 16
        for half in range(2):
            wv = 2 * s + half
            pltpu.sync_copy(id_v.at[pl.ds(half * EPW, EPW)],
                            srcp_out.at[wv, pl.ds(0, EPW)])
            pltpu.sync_copy(dump_v, srcp_out.at[wv, pl.ds(EPW, PW)])

    @pl.when(c == 1)
    def _():
        for g in range(PW // 16):
            dump_v[pl.ds(g * 16, 16)] = iota + (N + g * 16)
        for half in range(2):
            wv = 2 * s + half
            pltpu.sync_copy(id_v.at[pl.ds(half * EPW, EPW)],
                            dstp_out.at[wv, pl.ds(0, EPW)])
            pltpu.sync_copy(dump_v, dstp_out.at[wv, pl.ds(EPW, PW)])

    plsc.subcore_barrier()
    for cc in range(HR // 128):
        pltpu.sync_copy(h_v.at[pl.ds(cc * 128, 128)], sh.at[comb_v.at[cc]], add=True)
    plsc.subcore_barrier()

    pltpu.sync_copy(sh.at[pl.ds(s * rows, rows)], deg_v)
    half3 = jnp.full((16,), 1.5, jnp.float32)

    def nrm(i, carry):
        dg = deg_v[i, :]
        d = jnp.maximum(dg, 1.0)
        y = plsc.bitcast(jnp.full((16,), 0x5F3759DF, jnp.int32) -
                         (plsc.bitcast(d, jnp.int32) >> 1), jnp.float32)
        hd = 0.5 * d
        y = y * (half3 - hd * y * y)
        y = y * (half3 - hd * y * y)
        y = y * (half3 - hd * y * y)
        nrm_v[pl.ds(i * 16, 16)] = jnp.where(dg > 0, y, 0.0)
        return carry

    lax.fori_loop(0, rows, nrm, 0)

    @pl.when(c == 0)
    def _():
        pltpu.sync_copy(nrm_v, norm_s_out.at[pl.ds(s * (16 * rows), 16 * rows)])

    @pl.when(c == 1)
    def _():
        pltpu.sync_copy(nrm_v, norm_d_out.at[pl.ds(s * (16 * rows), 16 * rows)])


_deg_call = functools.partial(
    pl.kernel,
    out_type=[
        jax.ShapeDtypeStruct((NP,), jnp.float32),
        jax.ShapeDtypeStruct((NP,), jnp.float32),
        jax.ShapeDtypeStruct((NW, KCH), jnp.int32),
        jax.ShapeDtypeStruct((NW, KCH), jnp.int32),
    ],
    mesh=_MESH,
    compiler_params=_SC_PARAMS,
    scratch_types=[
        pltpu.VMEM((EPT,), jnp.int32),
        pltpu.VMEM((HR, 16), jnp.float32),
        pltpu.VMEM((HR // 128, 128), jnp.int32),
        pltpu.VMEM((HR // NS, 16), jnp.float32),
        pltpu.VMEM((16 * (HR // NS),), jnp.float32),
        pltpu.VMEM((PW,), jnp.int32),
        pltpu.VMEM_SHARED((HR, 16), jnp.float32),
    ],
)(_deg_body)


def _agg_body(xs_hbm, srcp_hbm, dstp_hbm, zer_hbm, out_hbm,
              src_v, dst_v, bufs, g0, g1, g2, s0, s1, s2, agg_sh):
    c = lax.axis_index("c")
    s = lax.axis_index("s")
    w = s * NC + c
    rows = NP // NS
    sems_g = [g0, g1, g2]
    sems_s = [s0, s1, s2]

    pltpu.sync_copy(zer_hbm.at[pl.ds(s * rows, rows)], agg_sh.at[pl.ds(s * rows, rows)])
    pltpu.sync_copy(srcp_hbm.at[w], src_v)
    pltpu.sync_copy(dstp_hbm.at[w], dst_v)
    plsc.subcore_barrier()

    def gather(t, b):
        pltpu.async_copy(xs_hbm.at[src_v.at[t]], bufs.at[b], sems_g[b])

    def scat_wait(t, b):
        pltpu.make_async_copy(bufs.at[b], agg_sh.at[dst_v.at[t]], sems_s[b]).wait()

    gather(0, 0)
    gather(1, 1)

    def outer(g, carry):
        for b in range(NB):
            t = g * NB + b
            pltpu.make_async_copy(xs_hbm.at[src_v.at[t]], bufs.at[b], sems_g[b]).wait()
            pltpu.async_copy(bufs.at[b], agg_sh.at[dst_v.at[t]], sems_s[b], add=True)
            b2 = (b + 2) % NB

            @pl.when(t >= 1)
            def _():
                scat_wait(t - 1, b2)

            @pl.when(t + 2 < K)
            def _():
                gather(t + 2, b2)

        return carry

    lax.fori_loop(0, K // NB, outer, 0)
    scat_wait(K - 1, (K - 1) % NB)
    plsc.subcore_barrier()

    pltpu.sync_copy(agg_sh.at[pl.ds(s * rows, rows)], out_hbm.at[c, pl.ds(s * rows, rows)])


_agg_call = functools.partial(
    pl.kernel,
    out_type=jax.ShapeDtypeStruct((NC, NP, D), jnp.float32),
    mesh=_MESH,
    compiler_params=_SC_PARAMS,
    scratch_types=[
        pltpu.VMEM((K, CH), jnp.int32),
        pltpu.VMEM((K, CH), jnp.int32),
        pltpu.VMEM((NB, CH, D), jnp.float32),
        pltpu.SemaphoreType.DMA,
        pltpu.SemaphoreType.DMA,
        pltpu.SemaphoreType.DMA,
        pltpu.SemaphoreType.DMA,
        pltpu.SemaphoreType.DMA,
        pltpu.SemaphoreType.DMA,
        pltpu.VMEM_SHARED((NP, D), jnp.float32),
    ],
)(_agg_body)


BR = 1000


def _prep_body(x_ref, ns_ref, xs_ref):
    xs_ref[...] = x_ref[...] * ns_ref[...]


_col = pl.BlockSpec((BR, 1), lambda i: (i, 0))
_rowblk = pl.BlockSpec((BR, D), lambda i: (i, 0))

_prep_call = pl.pallas_call(
    _prep_body,
    grid=(N // BR,),
    in_specs=[_rowblk, _col],
    out_specs=_rowblk,
    out_shape=jax.ShapeDtypeStruct((N, D), jnp.float32),
)


def _dense_body(a_ref, nd, ns, w_ref, b_ref, o_ref, *, final):
    g = (a_ref[0] + a_ref[1]) * nd[...]
    h = jnp.dot(g, w_ref[...], preferred_element_type=jnp.float32) + b_ref[...]
    h = jnp.maximum(h, 0.0)
    o_ref[...] = h if final else h * ns[...]


def _make_dense(final):
    return pl.pallas_call(
        functools.partial(_dense_body, final=final),
        grid=(N // BR,),
        in_specs=[
            pl.BlockSpec((NC, BR, D), lambda i: (0, i, 0)),
            _col, _col,
            pl.BlockSpec((D, D), lambda i: (0, 0)),
            pl.BlockSpec((1, D), lambda i: (0, 0)),
        ],
        out_specs=_rowblk,
        out_shape=jax.ShapeDtypeStruct((N, D), jnp.float32),
    )


_dense_mid = _make_dense(False)
_dense_fin = _make_dense(True)


def kernel(x, edge_index, W1, b1, W2, b2):
    ns_f, nd_f, srcp_f, dstp_f = _deg_call(edge_index.astype(jnp.int32))
    ns = ns_f.reshape(NP, 1)
    nd = nd_f.reshape(NP, 1)
    xs1 = _prep_call(x, ns)

    srcp = srcp_f.reshape(NW, K, CH)
    dstp = dstp_f.reshape(NW, K, CH)
    zer = jnp.zeros((NP, D), jnp.float32)

    agg1 = _agg_call(xs1, srcp, dstp, zer)
    h1s = _dense_mid(agg1, nd, ns, W1, b1.reshape(1, D))
    agg2 = _agg_call(h1s, srcp, dstp, zer)
    return _dense_fin(agg2, nd, ns, W2, b2.reshape(1, D))


# speedup vs baseline: 21.4797x; 1.0118x over previous
import functools

import jax
import jax.numpy as jnp
from jax import lax
from jax.experimental import pallas as pl
from jax.experimental.pallas import tpu as pltpu
from jax.experimental.pallas import tpu_sc as plsc

N = 10000
E = 320000
D = 128
NC = 2
NS = 16
NW = NC * NS
NP = 10240

HR = NP // 16
EPW = E // NW
EPT = E // NS
DEG_G = EPT // 16

CH = 64
NB = 3
K = 159
KCH = K * CH
PW = KCH - EPW

_MESH = plsc.VectorSubcoreMesh(core_axis_name="c", subcore_axis_name="s")
_SC_PARAMS = pltpu.CompilerParams(needs_layout_passes=False,
                                  use_tc_tiling_on_sc=False)


def _deg_body(ei_hbm, norm_s_out, norm_d_out, srcp_out, dstp_out,
              id_v, h_v, comb_v, deg_v, nrm_v, dump_v, sh):
    c = lax.axis_index("c")
    s = lax.axis_index("s")
    rows = HR // NS
    iota = lax.iota(jnp.int32, 16)
    z = jnp.zeros((16,), jnp.float32)

    def zero_row(i, carry):
        h_v[i, :] = z
        return carry

    lax.fori_loop(0, HR, zero_row, 0)
    pltpu.sync_copy(h_v.at[pl.ds(s * rows, rows)], sh.at[pl.ds(s * rows, rows)])

    @pl.when(c == 0)
    def _():
        pltpu.sync_copy(ei_hbm.at[0, pl.ds(s * EPT, EPT)], id_v)

    @pl.when(c == 1)
    def _():
        pltpu.sync_copy(ei_hbm.at[1, pl.ds(s * EPT, EPT)], id_v)
    for cc in range(HR // 128):
        for g in range(128 // 16):
            comb_v[cc, pl.ds(g * 16, 16)] = iota + (cc * 128 + g * 16)

    ones = jnp.ones((16,), jnp.float32)

    def scat(i, carry):
        ids = id_v[pl.ds(i * 16, 16)]
        plsc.addupdate_scatter(h_v, [ids >> 4, ids & 15], ones)
        return carry

    lax.fori_loop(0, DEG_G, scat, 0)

    @pl.when(c == 0)
    def _():
        for g in range(PW // 16):
            dump_v[pl.ds(g * 16, 16)] = iota + g * 16
        for half in range(2):
            wv = 2 * s + half
            pltpu.sync_copy(id_v.at[pl.ds(half * EPW, EPW)],
                            srcp_out.at[wv, pl.ds(0, EPW)])
            pltpu.sync_copy(dump_v, srcp_out.at[wv, pl.ds(EPW, PW)])

    @pl.when(c == 1)
    def _():
        for g in range(PW // 16):
            dump_v[pl.ds(g * 16, 16)] = iota + (N + g * 16)
        for half in range(2):
            wv = 2 * s + half
            pltpu.sync_copy(id_v.at[pl.ds(half * EPW, EPW)],
                            dstp_out.at[wv, pl.ds(0, EPW)])
            pltpu.sync_copy(dump_v, dstp_out.at[wv, pl.ds(EPW, PW)])

    plsc.subcore_barrier()
    for cc in range(HR // 128):
        pltpu.sync_copy(h_v.at[pl.ds(cc * 128, 128)], sh.at[comb_v.at[cc]], add=True)
    plsc.subcore_barrier()

    pltpu.sync_copy(sh.at[pl.ds(s * rows, rows)], deg_v)
    half3 = jnp.full((16,), 1.5, jnp.float32)

    def nrm(i, carry):
        dg = deg_v[i, :]
        d = jnp.maximum(dg, 1.0)
        y = plsc.bitcast(jnp.full((16,), 0x5F3759DF, jnp.int32) -
                         (plsc.bitcast(d, jnp.int32) >> 1), jnp.float32)
        hd = 0.5 * d
        y = y * (half3 - hd * y * y)
        y = y * (half3 - hd * y * y)
        y = y * (half3 - hd * y * y)
        nrm_v[pl.ds(i * 16, 16)] = jnp.where(dg > 0, y, 0.0)
        return carry

    lax.fori_loop(0, rows, nrm, 0)

    @pl.when(c == 0)
    def _():
        pltpu.sync_copy(nrm_v, norm_s_out.at[pl.ds(s * (16 * rows), 16 * rows)])

    @pl.when(c == 1)
    def _():
        pltpu.sync_copy(nrm_v, norm_d_out.at[pl.ds(s * (16 * rows), 16 * rows)])


_deg_call = functools.partial(
    pl.kernel,
    out_type=[
        jax.ShapeDtypeStruct((NP,), jnp.float32),
        jax.ShapeDtypeStruct((NP,), jnp.float32),
        jax.ShapeDtypeStruct((NW, KCH), jnp.int32),
        jax.ShapeDtypeStruct((NW, KCH), jnp.int32),
    ],
    mesh=_MESH,
    compiler_params=_SC_PARAMS,
    scratch_types=[
        pltpu.VMEM((EPT,), jnp.int32),
        pltpu.VMEM((HR, 16), jnp.float32),
        pltpu.VMEM((HR // 128, 128), jnp.int32),
        pltpu.VMEM((HR // NS, 16), jnp.float32),
        pltpu.VMEM((16 * (HR // NS),), jnp.float32),
        pltpu.VMEM((PW,), jnp.int32),
        pltpu.VMEM_SHARED((HR, 16), jnp.float32),
    ],
)(_deg_body)


def _agg_body(xs_hbm, srcp_hbm, dstp_hbm, out_hbm,
              src_v, dst_v, bufs, g0, g1, g2, s0, s1, s2, agg_sh):
    c = lax.axis_index("c")
    s = lax.axis_index("s")
    w = s * NC + c
    rows = NP // NS
    sems_g = [g0, g1, g2]
    sems_s = [s0, s1, s2]

    z = jnp.zeros((16,), jnp.float32)

    def zero_buf(i, carry):
        bufs[0, i >> 3, pl.ds((i & 7) * 16, 16)] = z
        return carry

    lax.fori_loop(0, CH * D // 16, zero_buf, 0)
    for r in range(rows // CH):
        pltpu.sync_copy(bufs.at[0], agg_sh.at[pl.ds(s * rows + r * CH, CH)])
    pltpu.sync_copy(srcp_hbm.at[w], src_v)
    pltpu.sync_copy(dstp_hbm.at[w], dst_v)
    plsc.subcore_barrier()

    def gather(t, b):
        pltpu.async_copy(xs_hbm.at[src_v.at[t]], bufs.at[b], sems_g[b])

    def scat_wait(t, b):
        pltpu.make_async_copy(bufs.at[b], agg_sh.at[dst_v.at[t]], sems_s[b]).wait()

    gather(0, 0)
    gather(1, 1)

    def outer(g, carry):
        for b in range(NB):
            t = g * NB + b
            pltpu.make_async_copy(xs_hbm.at[src_v.at[t]], bufs.at[b], sems_g[b]).wait()
            pltpu.async_copy(bufs.at[b], agg_sh.at[dst_v.at[t]], sems_s[b], add=True)
            b2 = (b + 2) % NB

            @pl.when(t >= 1)
            def _():
                scat_wait(t - 1, b2)

            @pl.when(t + 2 < K)
            def _():
                gather(t + 2, b2)

        return carry

    lax.fori_loop(0, K // NB, outer, 0)
    scat_wait(K - 1, (K - 1) % NB)
    plsc.subcore_barrier()

    pltpu.sync_copy(agg_sh.at[pl.ds(s * rows, rows)], out_hbm.at[c, pl.ds(s * rows, rows)])


_agg_call = functools.partial(
    pl.kernel,
    out_type=jax.ShapeDtypeStruct((NC, NP, D), jnp.float32),
    mesh=_MESH,
    compiler_params=_SC_PARAMS,
    scratch_types=[
        pltpu.VMEM((K, CH), jnp.int32),
        pltpu.VMEM((K, CH), jnp.int32),
        pltpu.VMEM((NB, CH, D), jnp.float32),
        pltpu.SemaphoreType.DMA,
        pltpu.SemaphoreType.DMA,
        pltpu.SemaphoreType.DMA,
        pltpu.SemaphoreType.DMA,
        pltpu.SemaphoreType.DMA,
        pltpu.SemaphoreType.DMA,
        pltpu.VMEM_SHARED((NP, D), jnp.float32),
    ],
)(_agg_body)


BR = 1000


def _prep_body(x_ref, ns_ref, xs_ref):
    xs_ref[...] = x_ref[...] * ns_ref[...]


_col = pl.BlockSpec((BR, 1), lambda i: (i, 0))
_rowblk = pl.BlockSpec((BR, D), lambda i: (i, 0))

_prep_call = pl.pallas_call(
    _prep_body,
    grid=(N // BR,),
    in_specs=[_rowblk, _col],
    out_specs=_rowblk,
    out_shape=jax.ShapeDtypeStruct((N, D), jnp.float32),
)


def _dense_body(a_ref, nd, ns, w_ref, b_ref, o_ref, *, final):
    g = (a_ref[0] + a_ref[1]) * nd[...]
    h = jnp.dot(g, w_ref[...], preferred_element_type=jnp.float32) + b_ref[...]
    h = jnp.maximum(h, 0.0)
    o_ref[...] = h if final else h * ns[...]


def _make_dense(final):
    return pl.pallas_call(
        functools.partial(_dense_body, final=final),
        grid=(N // BR,),
        in_specs=[
            pl.BlockSpec((NC, BR, D), lambda i: (0, i, 0)),
            _col, _col,
            pl.BlockSpec((D, D), lambda i: (0, 0)),
            pl.BlockSpec((1, D), lambda i: (0, 0)),
        ],
        out_specs=_rowblk,
        out_shape=jax.ShapeDtypeStruct((N, D), jnp.float32),
    )


_dense_mid = _make_dense(False)
_dense_fin = _make_dense(True)


def kernel(x, edge_index, W1, b1, W2, b2):
    ns_f, nd_f, srcp_f, dstp_f = _deg_call(edge_index.astype(jnp.int32))
    ns = ns_f.reshape(NP, 1)
    nd = nd_f.reshape(NP, 1)
    xs1 = _prep_call(x, ns)

    srcp = srcp_f.reshape(NW, K, CH)
    dstp = dstp_f.reshape(NW, K, CH)

    agg1 = _agg_call(xs1, srcp, dstp)
    h1s = _dense_mid(agg1, nd, ns, W1, b1.reshape(1, D))
    agg2 = _agg_call(h1s, srcp, dstp)
    return _dense_fin(agg2, nd, ns, W2, b2.reshape(1, D))
